# baseline (speedup 1.0000x reference)
import functools

import jax
import jax.numpy as jnp
from jax import lax
from jax.experimental import pallas as pl
from jax.experimental.pallas import tpu as pltpu
from jax.experimental.pallas import tpu_sc as plsc

_N, _T, _D = 4, 8192, 1024
_NW = 32
_C = 32
_ROWS_PER_W = (_N * _T) // _NW
_N_CHUNKS = _ROWS_PER_W // _C


def _sc_kernel(x_hbm, pos_hbm, out_hbm, xv, pv):
    wid = lax.axis_index("s") * 2 + lax.axis_index("c")
    row_base = wid * _ROWS_PER_W
    t_base = row_base % _T

    def chunk_body(i, carry):
        row0 = row_base + i * _C
        t0 = t_base + i * _C
        pltpu.sync_copy(x_hbm.at[pl.ds(row0, _C), :], xv)
        pltpu.sync_copy(pos_hbm.at[pl.ds(t0, _C), :], pv)

        def row_body(r, c2):
            for u in range(_D // 16):
                sl = pl.ds(u * 16, 16)
                xv[r, sl] = xv[r, sl] + pv[r, sl]
            return c2

        lax.fori_loop(0, _C, row_body, 0, unroll=False)
        pltpu.sync_copy(xv, out_hbm.at[pl.ds(row0, _C), :])
        return carry

    lax.fori_loop(0, _N_CHUNKS, chunk_body, 0, unroll=False)


@functools.partial(jax.jit, static_argnums=())
def _sc_add(Xf, pos_table):
    mesh = plsc.VectorSubcoreMesh(core_axis_name="c", subcore_axis_name="s")
    kfn = functools.partial(
        pl.kernel,
        mesh=mesh,
        out_type=jax.ShapeDtypeStruct((_N * _T, _D), jnp.float32),
        scratch_types=[
            pltpu.VMEM((_C, _D), jnp.float32),
            pltpu.VMEM((_C, _D), jnp.float32),
        ],
    )(_sc_kernel)
    return kfn(Xf, pos_table)


def kernel(X, pos_table):
    N, T, D = X.shape
    out = _sc_add(X.reshape(N * T, D), pos_table)
    return out.reshape(N, T, D)


# speedup vs baseline: 3.3271x; 3.3271x over previous
import jax
import jax.numpy as jnp
from jax.experimental import pallas as pl


_BLOCK_T = 2048


def _add_kernel(x_ref, pos_ref, o_ref):
    o_ref[...] = x_ref[...] + pos_ref[...]


def kernel(X, pos_table):
    N, T, D = X.shape
    bt = min(_BLOCK_T, T)
    grid = (T // bt, N)
    return pl.pallas_call(
        _add_kernel,
        grid=grid,
        in_specs=[
            pl.BlockSpec((1, bt, D), lambda t, n: (n, t, 0)),
            pl.BlockSpec((bt, D), lambda t, n: (t, 0)),
        ],
        out_specs=pl.BlockSpec((1, bt, D), lambda t, n: (n, t, 0)),
        out_shape=jax.ShapeDtypeStruct((N, T, D), X.dtype),
    )(X, pos_table)
